# baseline (speedup 1.0000x reference)
import functools

import jax
import jax.numpy as jnp
from jax import lax
from jax.experimental import pallas as pl
from jax.experimental.pallas import tpu as pltpu
from jax.experimental.pallas import tpu_sc as plsc

_CHUNK_ROWS = 32
_CORE0_PER8 = 5


def _make_copy_kernel(seq_len: int, d_model: int):
    info = plsc.get_sparse_core_info()
    nc, ns = info.num_cores, info.num_subcores
    total_chunks = seq_len // _CHUNK_ROWS
    ngroups = total_chunks // 8
    core_chunks = [[], []]
    for grp in range(ngroups):
        for k in range(8):
            core_chunks[0 if k < _CORE0_PER8 else 1].append(grp * 8 + k)
    mesh = plsc.VectorSubcoreMesh(core_axis_name="c", subcore_axis_name="s")

    @functools.partial(
        pl.kernel,
        out_type=jax.ShapeDtypeStruct((seq_len, d_model), jnp.float32),
        mesh=mesh,
        scratch_types=[
            pltpu.VMEM((_CHUNK_ROWS, d_model), jnp.float32),
            pltpu.VMEM((_CHUNK_ROWS, d_model), jnp.float32),
            pltpu.SemaphoreType.DMA,
            pltpu.SemaphoreType.DMA,
            pltpu.SemaphoreType.DMA,
            pltpu.SemaphoreType.DMA,
        ],
    )
    def copy_kernel(table_hbm, out_hbm, buf0, buf1, si0, si1, so0, so1):
        cid = lax.axis_index("c")
        sid = lax.axis_index("s")
        bufs = (buf0, buf1)
        in_sems = (si0, si1)
        out_sems = (so0, so1)

        def run(core, chunks):
            per_w = len(chunks) // ns
            K = _CORE0_PER8 if core == 0 else 8 - _CORE0_PER8
            off = 0 if core == 0 else _CORE0_PER8

            def base_row(j):
                i = sid * per_w + j
                return ((i // K) * 8 + (i % K) + off) * _CHUNK_ROWS

            def chunk_src(j):
                return table_hbm.at[pl.ds(base_row(j), _CHUNK_ROWS)]

            def chunk_dst(j):
                return out_hbm.at[pl.ds(base_row(j), _CHUNK_ROWS)]

            nchunks = per_w
            in_copies = [None] * nchunks
            out_copies = [None] * nchunks
            in_copies[0] = pltpu.async_copy(chunk_src(0), bufs[0], in_sems[0])
            for c in range(nchunks):
                b = c % 2
                in_copies[c].wait()
                out_copies[c] = pltpu.async_copy(bufs[b], chunk_dst(c), out_sems[b])
                if c + 1 < nchunks:
                    if c >= 1:
                        out_copies[c - 1].wait()
                    nb = (c + 1) % 2
                    in_copies[c + 1] = pltpu.async_copy(
                        chunk_src(c + 1), bufs[nb], in_sems[nb]
                    )
            if nchunks >= 2:
                out_copies[nchunks - 2].wait()
            out_copies[nchunks - 1].wait()

        @pl.when(cid == 0)
        def _():
            run(0, core_chunks[0])

        @pl.when(cid == 1)
        def _():
            run(1, core_chunks[1])

    return copy_kernel


def kernel(x, table):
    seq_len = x.shape[1]
    return _make_copy_kernel(seq_len, table.shape[1])(table)
